# baseline (speedup 1.0000x reference)
import jax
import jax.numpy as jnp
from jax import lax
from jax.experimental import pallas as pl
from jax.experimental.pallas import tpu as pltpu
from jax.experimental.pallas import tpu_sc as plsc

NUM_HEADS = 16
EMBED = 3969
ROWS = 1024
COLS = 1024
LANES = 16
NUM_WORKERS = 32
ROWS_PER_TILE = ROWS // NUM_WORKERS


def _gather_body(table_hbm, idx_hbm, out_hbm, table_v, idx_v0, idx_v1,
                 out_v0, out_v1, sem_tab, sem_in0, sem_in1, sem_out0,
                 sem_out1):
    idx_bufs = (idx_v0, idx_v1)
    out_bufs = (out_v0, out_v1)
    sems_in = (sem_in0, sem_in1)
    sems_out = (sem_out0, sem_out1)
    wid = lax.axis_index("s") * 2 + lax.axis_index("c")
    base = wid * ROWS_PER_TILE

    pltpu.async_copy(table_hbm, table_v, sem_tab).wait()

    pltpu.make_async_copy(
        idx_hbm.at[pl.ds(base, 1)], idx_bufs[0], sems_in[0]
    ).start()

    def row_pair(r2, carry):
        for b in range(2):
            r = r2 * 2 + b

            @pl.when(r + 1 < ROWS_PER_TILE)
            def _prefetch():
                pltpu.make_async_copy(
                    idx_hbm.at[pl.ds(base + r + 1, 1)],
                    idx_bufs[1 - b],
                    sems_in[1 - b],
                ).start()

            pltpu.make_async_copy(
                idx_hbm.at[pl.ds(base + r, 1)], idx_bufs[b], sems_in[b]
            ).wait()

            @pl.when(r2 >= 1)
            def _drain():
                pltpu.make_async_copy(
                    out_bufs[b],
                    out_hbm.at[:, pl.ds(base + r, 1), :],
                    sems_out[b],
                ).wait()

            @plsc.parallel_loop(0, COLS, step=LANES, unroll=8)
            def _cols(col):
                iv = idx_bufs[b][0, pl.ds(col, LANES)]
                for h in range(NUM_HEADS):
                    vals = plsc.load_gather(table_v, [iv + h * EMBED])
                    out_bufs[b][h, 0, pl.ds(col, LANES)] = vals

            pltpu.make_async_copy(
                out_bufs[b],
                out_hbm.at[:, pl.ds(base + r, 1), :],
                sems_out[b],
            ).start()
        return carry

    lax.fori_loop(0, ROWS_PER_TILE // 2, row_pair, 0)

    for b in range(2):
        pltpu.make_async_copy(
            out_bufs[b],
            out_hbm.at[:, pl.ds(base + b, 1), :],
            sems_out[b],
        ).wait()


@jax.jit
def _rpe_gather(idx, table_flat):
    mesh = plsc.VectorSubcoreMesh(core_axis_name="c", subcore_axis_name="s")
    run = pl.kernel(
        _gather_body,
        out_type=jax.ShapeDtypeStruct((NUM_HEADS, ROWS, COLS), jnp.float32),
        mesh=mesh,
        compiler_params=pltpu.CompilerParams(needs_layout_passes=False),
        scratch_types=[
            pltpu.VMEM((NUM_HEADS * EMBED,), jnp.float32),
            pltpu.VMEM((1, COLS), jnp.int32),
            pltpu.VMEM((1, COLS), jnp.int32),
            pltpu.VMEM((NUM_HEADS, 1, COLS), jnp.float32),
            pltpu.VMEM((NUM_HEADS, 1, COLS), jnp.float32),
            pltpu.SemaphoreType.DMA,
            pltpu.SemaphoreType.DMA,
            pltpu.SemaphoreType.DMA,
            pltpu.SemaphoreType.DMA,
            pltpu.SemaphoreType.DMA,
        ],
    )
    return run(table_flat, idx)


def kernel(attn_rpe_index, relative_position_bias_table):
    idx = attn_rpe_index.astype(jnp.int32)
    table_flat = relative_position_bias_table.reshape(-1)
    out = _rpe_gather(idx, table_flat)
    return out[None]


# speedup vs baseline: 1.1109x; 1.1109x over previous
import jax
import jax.numpy as jnp
from jax import lax
from jax.experimental import pallas as pl
from jax.experimental.pallas import tpu as pltpu
from jax.experimental.pallas import tpu_sc as plsc

NUM_HEADS = 16
EMBED = 3969
ROWS = 1024
COLS = 1024
LANES = 16
NUM_WORKERS = 32
ROWS_PER_TILE = ROWS // NUM_WORKERS


def _gather_body(table_hbm, idx_hbm, out_hbm, table_v, idx_v0, idx_v1,
                 out_v0, out_v1, sem_tab, sem_in0, sem_in1, sem_out0,
                 sem_out1):
    idx_bufs = (idx_v0, idx_v1)
    out_bufs = (out_v0, out_v1)
    sems_in = (sem_in0, sem_in1)
    sems_out = (sem_out0, sem_out1)
    wid = lax.axis_index("s") * 2 + lax.axis_index("c")
    base = wid * ROWS_PER_TILE

    pltpu.async_copy(table_hbm, table_v, sem_tab).wait()

    pltpu.make_async_copy(
        idx_hbm.at[pl.ds(base, 1)], idx_bufs[0], sems_in[0]
    ).start()

    def row_pair(r2, carry):
        for b in range(2):
            r = r2 * 2 + b

            @pl.when(r + 1 < ROWS_PER_TILE)
            def _prefetch():
                pltpu.make_async_copy(
                    idx_hbm.at[pl.ds(base + r + 1, 1)],
                    idx_bufs[1 - b],
                    sems_in[1 - b],
                ).start()

            pltpu.make_async_copy(
                idx_hbm.at[pl.ds(base + r, 1)], idx_bufs[b], sems_in[b]
            ).wait()

            @pl.when(r2 >= 1)
            def _drain():
                pltpu.make_async_copy(
                    out_bufs[b],
                    out_hbm.at[:, pl.ds(base + r, 1), :],
                    sems_out[b],
                ).wait()

            @plsc.parallel_loop(0, COLS, step=LANES, unroll=2)
            def _cols(col):
                iv = idx_bufs[b][0, pl.ds(col, LANES)]
                for h in range(NUM_HEADS):
                    vals = plsc.load_gather(table_v, [iv + h * EMBED])
                    out_bufs[b][h, 0, pl.ds(col, LANES)] = vals

            pltpu.make_async_copy(
                out_bufs[b],
                out_hbm.at[:, pl.ds(base + r, 1), :],
                sems_out[b],
            ).start()
        return carry

    lax.fori_loop(0, ROWS_PER_TILE // 2, row_pair, 0)

    for b in range(2):
        pltpu.make_async_copy(
            out_bufs[b],
            out_hbm.at[:, pl.ds(base + b, 1), :],
            sems_out[b],
        ).wait()


@jax.jit
def _rpe_gather(idx, table_flat):
    mesh = plsc.VectorSubcoreMesh(core_axis_name="c", subcore_axis_name="s")
    run = pl.kernel(
        _gather_body,
        out_type=jax.ShapeDtypeStruct((NUM_HEADS, ROWS, COLS), jnp.float32),
        mesh=mesh,
        compiler_params=pltpu.CompilerParams(needs_layout_passes=False),
        scratch_types=[
            pltpu.VMEM((NUM_HEADS * EMBED,), jnp.float32),
            pltpu.VMEM((1, COLS), jnp.int32),
            pltpu.VMEM((1, COLS), jnp.int32),
            pltpu.VMEM((NUM_HEADS, 1, COLS), jnp.float32),
            pltpu.VMEM((NUM_HEADS, 1, COLS), jnp.float32),
            pltpu.SemaphoreType.DMA,
            pltpu.SemaphoreType.DMA,
            pltpu.SemaphoreType.DMA,
            pltpu.SemaphoreType.DMA,
            pltpu.SemaphoreType.DMA,
        ],
    )
    return run(table_flat, idx)


def kernel(attn_rpe_index, relative_position_bias_table):
    idx = attn_rpe_index.astype(jnp.int32)
    table_flat = relative_position_bias_table.reshape(-1)
    out = _rpe_gather(idx, table_flat)
    return out[None]


# speedup vs baseline: 1.1513x; 1.0363x over previous
import jax
import jax.numpy as jnp
from jax import lax
from jax.experimental import pallas as pl
from jax.experimental.pallas import tpu as pltpu
from jax.experimental.pallas import tpu_sc as plsc

NUM_HEADS = 16
EMBED = 3969
ROWS = 1024
COLS = 1024
LANES = 16
HEAD_GROUPS = 2
HEADS_PER_TILE = NUM_HEADS // HEAD_GROUPS
ROW_GROUPS = 16
ROWS_PER_TILE = ROWS // ROW_GROUPS
CHUNK_ROWS = 4
CHUNKS = ROWS_PER_TILE // CHUNK_ROWS


def _gather_body(table_hbm, idx_hbm, out_hbm, table_v, idx_v0, idx_v1,
                 out_v0, out_v1, sem_tab, sem_in0, sem_in1, sem_out0,
                 sem_out1):
    idx_bufs = (idx_v0, idx_v1)
    out_bufs = (out_v0, out_v1)
    sems_in = (sem_in0, sem_in1)
    sems_out = (sem_out0, sem_out1)
    wid = lax.axis_index("s") * 2 + lax.axis_index("c")
    hg = wid % HEAD_GROUPS
    rg = wid // HEAD_GROUPS
    base = rg * ROWS_PER_TILE
    head0 = hg * HEADS_PER_TILE

    pltpu.async_copy(
        table_hbm.at[pl.ds(head0 * EMBED, HEADS_PER_TILE * EMBED)],
        table_v, sem_tab,
    ).wait()

    pltpu.make_async_copy(
        idx_hbm.at[pl.ds(base, CHUNK_ROWS)], idx_bufs[0], sems_in[0]
    ).start()

    def chunk_pair(c2, carry):
        for b in range(2):
            c = c2 * 2 + b
            row = base + c * CHUNK_ROWS

            @pl.when(c + 1 < CHUNKS)
            def _prefetch():
                pltpu.make_async_copy(
                    idx_hbm.at[pl.ds(row + CHUNK_ROWS, CHUNK_ROWS)],
                    idx_bufs[1 - b],
                    sems_in[1 - b],
                ).start()

            pltpu.make_async_copy(
                idx_hbm.at[pl.ds(row, CHUNK_ROWS)], idx_bufs[b], sems_in[b]
            ).wait()

            @pl.when(c2 >= 1)
            def _drain():
                pltpu.make_async_copy(
                    out_bufs[b],
                    out_hbm.at[pl.ds(head0, HEADS_PER_TILE),
                               pl.ds(row, CHUNK_ROWS), :],
                    sems_out[b],
                ).wait()

            for rr in range(CHUNK_ROWS):
                @plsc.parallel_loop(0, COLS, step=LANES, unroll=2)
                def _cols(col):
                    iv = idx_bufs[b][rr, pl.ds(col, LANES)]
                    for h in range(HEADS_PER_TILE):
                        vals = plsc.load_gather(table_v, [iv + h * EMBED])
                        out_bufs[b][h, rr, pl.ds(col, LANES)] = vals

            pltpu.make_async_copy(
                out_bufs[b],
                out_hbm.at[pl.ds(head0, HEADS_PER_TILE),
                           pl.ds(row, CHUNK_ROWS), :],
                sems_out[b],
            ).start()
        return carry

    lax.fori_loop(0, CHUNKS // 2, chunk_pair, 0)

    for b in range(2):
        pltpu.make_async_copy(
            out_bufs[b],
            out_hbm.at[pl.ds(head0, HEADS_PER_TILE),
                       pl.ds(base, CHUNK_ROWS), :],
            sems_out[b],
        ).wait()


@jax.jit
def _rpe_gather(idx, table_flat):
    mesh = plsc.VectorSubcoreMesh(core_axis_name="c", subcore_axis_name="s")
    run = pl.kernel(
        _gather_body,
        out_type=jax.ShapeDtypeStruct((NUM_HEADS, ROWS, COLS), jnp.float32),
        mesh=mesh,
        compiler_params=pltpu.CompilerParams(needs_layout_passes=False),
        scratch_types=[
            pltpu.VMEM((HEADS_PER_TILE * EMBED,), jnp.float32),
            pltpu.VMEM((CHUNK_ROWS, COLS), jnp.int32),
            pltpu.VMEM((CHUNK_ROWS, COLS), jnp.int32),
            pltpu.VMEM((HEADS_PER_TILE, CHUNK_ROWS, COLS), jnp.float32),
            pltpu.VMEM((HEADS_PER_TILE, CHUNK_ROWS, COLS), jnp.float32),
            pltpu.SemaphoreType.DMA,
            pltpu.SemaphoreType.DMA,
            pltpu.SemaphoreType.DMA,
            pltpu.SemaphoreType.DMA,
            pltpu.SemaphoreType.DMA,
        ],
    )
    return run(table_flat, idx)


def kernel(attn_rpe_index, relative_position_bias_table):
    idx = attn_rpe_index.astype(jnp.int32)
    table_flat = relative_position_bias_table.reshape(-1)
    out = _rpe_gather(idx, table_flat)
    return out[None]


# speedup vs baseline: 1.1648x; 1.0118x over previous
import jax
import jax.numpy as jnp
from jax import lax
from jax.experimental import pallas as pl
from jax.experimental.pallas import tpu as pltpu
from jax.experimental.pallas import tpu_sc as plsc

NUM_HEADS = 16
EMBED = 3969
ROWS = 1024
COLS = 1024
LANES = 16
HEAD_GROUPS = 2
HEADS_PER_TILE = NUM_HEADS // HEAD_GROUPS
ROW_GROUPS = 16
ROWS_PER_TILE = ROWS // ROW_GROUPS
CHUNK_ROWS = 4
CHUNKS = ROWS_PER_TILE // CHUNK_ROWS


def _gather_body(table_hbm, idx_hbm, out_hbm, table_v, idx_v0, idx_v1,
                 out_v0, out_v1, sem_tab, sem_in0, sem_in1, sem_out0,
                 sem_out1):
    idx_bufs = (idx_v0, idx_v1)
    out_bufs = (out_v0, out_v1)
    sems_in = (sem_in0, sem_in1)
    sems_out = (sem_out0, sem_out1)
    wid = lax.axis_index("s") * 2 + lax.axis_index("c")
    hg = wid % HEAD_GROUPS
    rg = wid // HEAD_GROUPS
    base = rg * ROWS_PER_TILE
    head0 = hg * HEADS_PER_TILE

    pltpu.async_copy(
        table_hbm.at[pl.ds(head0 * EMBED, HEADS_PER_TILE * EMBED)],
        table_v, sem_tab,
    ).wait()

    pltpu.make_async_copy(
        idx_hbm.at[pl.ds(base, CHUNK_ROWS)], idx_bufs[0], sems_in[0]
    ).start()

    def chunk_pair(c2, carry):
        for b in range(2):
            c = c2 * 2 + b
            row = base + c * CHUNK_ROWS

            @pl.when(c + 1 < CHUNKS)
            def _prefetch():
                pltpu.make_async_copy(
                    idx_hbm.at[pl.ds(row + CHUNK_ROWS, CHUNK_ROWS)],
                    idx_bufs[1 - b],
                    sems_in[1 - b],
                ).start()

            pltpu.make_async_copy(
                idx_hbm.at[pl.ds(row, CHUNK_ROWS)], idx_bufs[b], sems_in[b]
            ).wait()

            @pl.when(c2 >= 1)
            def _drain():
                pltpu.make_async_copy(
                    out_bufs[b],
                    out_hbm.at[pl.ds(head0, HEADS_PER_TILE),
                               pl.ds(row, CHUNK_ROWS), :],
                    sems_out[b],
                ).wait()

            for rr in range(CHUNK_ROWS):
                @plsc.parallel_loop(0, COLS, step=LANES, unroll=4)
                def _cols(col):
                    iv = idx_bufs[b][rr, pl.ds(col, LANES)]
                    for h in range(HEADS_PER_TILE):
                        vals = plsc.load_gather(table_v, [iv + h * EMBED])
                        out_bufs[b][h, rr, pl.ds(col, LANES)] = vals

            pltpu.make_async_copy(
                out_bufs[b],
                out_hbm.at[pl.ds(head0, HEADS_PER_TILE),
                           pl.ds(row, CHUNK_ROWS), :],
                sems_out[b],
            ).start()
        return carry

    lax.fori_loop(0, CHUNKS // 2, chunk_pair, 0)

    for b in range(2):
        pltpu.make_async_copy(
            out_bufs[b],
            out_hbm.at[pl.ds(head0, HEADS_PER_TILE),
                       pl.ds(base, CHUNK_ROWS), :],
            sems_out[b],
        ).wait()


@jax.jit
def _rpe_gather(idx, table_flat):
    mesh = plsc.VectorSubcoreMesh(core_axis_name="c", subcore_axis_name="s")
    run = pl.kernel(
        _gather_body,
        out_type=jax.ShapeDtypeStruct((NUM_HEADS, ROWS, COLS), jnp.float32),
        mesh=mesh,
        compiler_params=pltpu.CompilerParams(needs_layout_passes=False),
        scratch_types=[
            pltpu.VMEM((HEADS_PER_TILE * EMBED,), jnp.float32),
            pltpu.VMEM((CHUNK_ROWS, COLS), jnp.int32),
            pltpu.VMEM((CHUNK_ROWS, COLS), jnp.int32),
            pltpu.VMEM((HEADS_PER_TILE, CHUNK_ROWS, COLS), jnp.float32),
            pltpu.VMEM((HEADS_PER_TILE, CHUNK_ROWS, COLS), jnp.float32),
            pltpu.SemaphoreType.DMA,
            pltpu.SemaphoreType.DMA,
            pltpu.SemaphoreType.DMA,
            pltpu.SemaphoreType.DMA,
            pltpu.SemaphoreType.DMA,
        ],
    )
    return run(table_flat, idx)


def kernel(attn_rpe_index, relative_position_bias_table):
    idx = attn_rpe_index.astype(jnp.int32)
    table_flat = relative_position_bias_table.reshape(-1)
    out = _rpe_gather(idx, table_flat)
    return out[None]
